# baseline (speedup 1.0000x reference)
import jax
import jax.numpy as jnp
from jax.experimental import pallas as pl
from jax.experimental.pallas import tpu as pltpu
from jax.experimental.pallas import tpu_sc as plsc

_W = 64
_IDX_BLK = 1024
_NTILES = 32


def kernel(indices, weight):
    B, L = indices.shape
    V, D = weight.shape
    N = B * L
    flat_idx = indices.reshape(N)

    rows_per_tile = N // _NTILES
    blocks_per_tile = rows_per_tile // _IDX_BLK
    wpb = _IDX_BLK // _W

    mesh = plsc.VectorSubcoreMesh(core_axis_name="core",
                                  subcore_axis_name="subcore")

    @pl.kernel(
        out_type=jax.ShapeDtypeStruct((N, D), weight.dtype),
        mesh=mesh,
        scratch_types=[
            pltpu.VMEM((2, _IDX_BLK), jnp.int32),
            pltpu.VMEM((2, _W, D), weight.dtype),
            pltpu.SemaphoreType.DMA,
            pltpu.SemaphoreType.DMA,
            pltpu.SemaphoreType.DMA,
            pltpu.SemaphoreType.DMA,
            pltpu.SemaphoreType.DMA,
            pltpu.SemaphoreType.DMA,
        ],
    )
    def sc_gather(i_hbm, w_hbm, o_hbm, idxb, rows,
                  isem0, isem1, gsem0, gsem1, wsem0, wsem1):
        isems = [isem0, isem1]
        gsems = [gsem0, gsem1]
        wsems = [wsem0, wsem1]

        wid = (jax.lax.axis_index("subcore") * 2
               + jax.lax.axis_index("core"))
        base = wid * rows_per_tile

        def wait_write(b, row0):
            pltpu.make_async_copy(rows.at[b], o_hbm.at[pl.ds(row0, _W)],
                                  wsems[b]).wait()

        def start_gather(p, k, b):
            pltpu.async_copy(w_hbm.at[idxb.at[p, pl.ds(k * _W, _W)]],
                             rows.at[b], gsems[b])

        pltpu.async_copy(i_hbm.at[pl.ds(base, _IDX_BLK)], idxb.at[0],
                         isems[0])
        pltpu.make_async_copy(i_hbm.at[pl.ds(base, _IDX_BLK)], idxb.at[0],
                              isems[0]).wait()
        start_gather(0, 0, 0)

        @pl.loop(0, blocks_per_tile, step=2)
        def _(g):
            for p in range(2):
                blk = g + p
                blk_base = base + blk * _IDX_BLK

                @pl.when(blk + 1 < blocks_per_tile)
                def _():
                    pltpu.async_copy(
                        i_hbm.at[pl.ds(blk_base + _IDX_BLK, _IDX_BLK)],
                        idxb.at[1 - p], isems[1 - p])

                @pl.loop(0, wpb, step=2)
                def _(kk):
                    for b in range(2):
                        k = kk + b
                        w_glob = blk * wpb + k
                        row0 = blk_base + k * _W

                        @pl.when(k + 1 < wpb)
                        def _():
                            @pl.when(w_glob + 1 >= 2)
                            def _():
                                wait_write(1 - b, row0 + _W)
                            start_gather(p, k + 1, 1 - b)

                        pltpu.make_async_copy(
                            w_hbm.at[idxb.at[p, pl.ds(k * _W, _W)]],
                            rows.at[b], gsems[b]).wait()
                        pltpu.async_copy(rows.at[b],
                                         o_hbm.at[pl.ds(row0, _W)],
                                         wsems[b])

                @pl.when(blk + 1 < blocks_per_tile)
                def _():
                    nxt = 1 - p
                    pltpu.make_async_copy(
                        i_hbm.at[pl.ds(blk_base + _IDX_BLK, _IDX_BLK)],
                        idxb.at[nxt], isems[nxt]).wait()
                    wait_write(0, blk_base + _IDX_BLK)
                    start_gather(nxt, 0, 0)

        for b in range(2):
            wait_write(b, base)

    out = sc_gather(flat_idx, weight)
    return out.reshape(B, L, D)
